# baseline (speedup 1.0000x reference)
import functools

import jax
import jax.numpy as jnp
from jax import lax
from jax.experimental import pallas as pl
from jax.experimental.pallas import tpu as pltpu
from jax.experimental.pallas import tpu_sc as plsc

N = 10000
E = 320000
IN, HID, CLS = 128, 256, 64

NTILES = 16
N_PAD = 10240
E_PAD = 327680
ZR = 32
RPT = N_PAD // NTILES


def _seg_sum_sc(dh, sub, ch, dtype=jnp.float32):
    assert ch % 2 == 0
    mesh = plsc.VectorSubcoreMesh(
        core_axis_name="c", subcore_axis_name="s", num_cores=2,
        num_subcores=NTILES,
    )

    @functools.partial(
        pl.kernel,
        out_type=jax.ShapeDtypeStruct((2 * N_PAD, dh), dtype),
        mesh=mesh,
        compiler_params=pltpu.CompilerParams(use_tc_tiling_on_sc=False),
        scratch_types=[
            pltpu.VMEM((2, sub * 128), jnp.int32),
            pltpu.VMEM((sub * 128,), jnp.int32),
            pltpu.VMEM((sub * 128,), jnp.int32),
            pltpu.VMEM((2, sub * 128, dh), dtype),
            pltpu.VMEM((ZR, dh), dtype),
            pltpu.SemaphoreType.DMA,
            pltpu.SemaphoreType.DMA,
            pltpu.VMEM_SHARED((N_PAD, dh), dtype),
        ],
    )
    def seg_sum(table, src3, dst3, zrows, out, sidx, didx0, didx1, rows, zbuf,
                sem0, sem1, acc):
        c = lax.axis_index("c")
        s = lax.axis_index("s")
        sems = (sem0, sem1)
        didxs = (didx0, didx1)

        pltpu.sync_copy(zrows, zbuf)
        for t in range(RPT // ZR):
            pltpu.sync_copy(zbuf, acc.at[pl.ds(s * RPT + t * ZR, ZR)])
        plsc.subcore_barrier()

        off = c * N_PAD

        def stage(ci, b):
            t = s * (ch + 2) + ci
            pltpu.sync_copy(src3.at[t], sidx.at[b])
            pltpu.sync_copy(dst3.at[t], didxs[b])
            for i in range(sub * 8):
                sidx[b, pl.ds(i * 16, 16)] = sidx[b, pl.ds(i * 16, 16)] + off
            pltpu.async_copy(table.at[sidx.at[b]], rows.at[b], sems[b])

        def drain_scatter(ci, b):
            pltpu.make_async_copy(
                table.at[sidx.at[b]], rows.at[b], sems[b]
            ).wait()
            pltpu.sync_copy(rows.at[b], acc.at[didxs[b]], add=True)
            stage(ci, b)

        stage(0, 0)
        stage(1, 1)

        def body2(i2, carry):
            ci = 2 * i2
            drain_scatter(ci + 2, 0)
            drain_scatter(ci + 3, 1)
            return carry

        lax.fori_loop(0, ch // 2, body2, 0)
        for b in range(2):
            pltpu.make_async_copy(
                table.at[sidx.at[b]], rows.at[b], sems[b]
            ).wait()
        plsc.subcore_barrier()
        pltpu.sync_copy(
            acc.at[pl.ds(s * RPT, RPT)], out.at[pl.ds(off + s * RPT, RPT)]
        )

    return seg_sum


_R = 1024


def _tc0_body(x_r, a0_r, a1_r, ws_r, wa_r, wb_r, b_r, o0_r, o1_r):
    inv = 1.0 / jnp.maximum(a0_r[:, 64:65].astype(jnp.float32), 1.0)
    z = jnp.dot(x_r[...], ws_r[...], preferred_element_type=jnp.float32)
    z = z + jnp.dot(a0_r[:, :64].astype(jnp.float32) * inv, wa_r[...],
                    preferred_element_type=jnp.float32)
    z = z + jnp.dot(a1_r[:, :64].astype(jnp.float32) * inv, wb_r[...],
                    preferred_element_type=jnp.float32)
    z = jnp.maximum(z + b_r[...], 0.0)
    o0_r[...] = z[:, :128].astype(jnp.bfloat16)
    o1_r[...] = z[:, 128:].astype(jnp.bfloat16)


def _tc1_body(h0_r, h1_r, a0_r, a1_r, dg_r, wsa_r, wsb_r, wna_r, wnb_r, b_r,
              wn2_r, h2_r, p0_r, p1_r):
    inv = 1.0 / jnp.maximum(dg_r[:, :1].astype(jnp.float32), 1.0)
    z = jnp.dot(h0_r[...], wsa_r[...], preferred_element_type=jnp.float32)
    z = z + jnp.dot(h1_r[...], wsb_r[...], preferred_element_type=jnp.float32)
    z = z + jnp.dot(a0_r[...] * inv, wna_r[...], preferred_element_type=jnp.float32)
    z = z + jnp.dot(a1_r[...] * inv, wnb_r[...], preferred_element_type=jnp.float32)
    z = jnp.maximum(z + b_r[...], 0.0)
    h2_r[...] = z
    p = jnp.dot(z, wn2_r[...], preferred_element_type=jnp.float32)
    p0_r[...] = p[:, :32].astype(jnp.bfloat16)
    p1_r[...] = p[:, 32:].astype(jnp.bfloat16)


def _tc2_body(h2_r, a0_r, a1_r, dg_r, ws_r, b_r, o_r):
    inv = 1.0 / jnp.maximum(dg_r[:, :1].astype(jnp.float32), 1.0)
    m = jnp.concatenate([a0_r[...] * inv, a1_r[...] * inv], axis=1)
    z = jnp.dot(h2_r[...], ws_r[...], preferred_element_type=jnp.float32)
    o_r[...] = z + m + b_r[...]


def _row_spec(w):
    return pl.BlockSpec((_R, w), lambda i: (i, 0))


def _full_spec(h, w):
    return pl.BlockSpec((h, w), lambda i: (0, 0))


def kernel(x, edge_index, W_self0, W_neigh0, b0, W_self1, W_neigh1, b1,
           W_self2, W_neigh2, b2):
    f32 = jnp.float32
    src = edge_index[0]
    dst = edge_index[1]
    padlen = E_PAD - E
    padv = jnp.full((padlen,), N_PAD - 1, jnp.int32)
    srcf = jnp.concatenate([src, padv])
    dstf = jnp.concatenate([dst, padv])

    def _chunked(flat, sub, ch):
        arr = flat.reshape(NTILES, ch, sub * 128)
        padc = jnp.full((NTILES, 2, sub * 128), N_PAD - 1, jnp.int32)
        return jnp.concatenate([arr, padc], axis=1).reshape(-1, sub * 128)

    src0, dst0 = _chunked(srcf, 4, 40), _chunked(dstf, 4, 40)
    src1, dst1 = _chunked(srcf, 4, 40), _chunked(dstf, 4, 40)
    srcp, dstp = _chunked(srcf, 16, 10), _chunked(dstf, 16, 10)

    bf16 = jnp.bfloat16
    xp = jnp.zeros((N_PAD, IN), f32).at[:N].set(x)
    xb = xp.astype(bf16)
    onescol = jnp.zeros((N_PAD, 16), bf16).at[:N].set(1.0)
    zerocol = jnp.zeros((N_PAD, 16), bf16)
    table0 = jnp.concatenate(
        [
            jnp.concatenate([xb[:, :64], onescol], axis=1),
            jnp.concatenate([xb[:, 64:], zerocol], axis=1),
        ],
        axis=0,
    )

    def _zr(dh):
        return jnp.zeros((ZR, dh), bf16)

    agg0 = _seg_sum_sc(80, 4, 40, bf16)(table0, src0, dst0, _zr(80))
    degs = agg0[:N_PAD, 64:80]

    nb = N_PAD // _R
    h1a, h1b = pl.pallas_call(
        _tc0_body,
        grid=(nb,),
        in_specs=[
            _row_spec(IN), _row_spec(80), _row_spec(80),
            _full_spec(IN, HID), _full_spec(64, HID), _full_spec(64, HID),
            _full_spec(1, HID),
        ],
        out_specs=[_row_spec(128), _row_spec(128)],
        out_shape=[
            jax.ShapeDtypeStruct((N_PAD, 128), jnp.bfloat16),
            jax.ShapeDtypeStruct((N_PAD, 128), jnp.bfloat16),
        ],
    )(xp, agg0[:N_PAD], agg0[N_PAD:], W_self0, W_neigh0[:64], W_neigh0[64:],
      b0.reshape(1, HID))

    table1 = jnp.concatenate([h1a, h1b], axis=0)
    agg1 = _seg_sum_sc(128, 4, 40, bf16)(table1, src1, dst1, _zr(128))

    h2, pa, pb = pl.pallas_call(
        _tc1_body,
        grid=(nb,),
        in_specs=[
            _row_spec(128), _row_spec(128), _row_spec(128), _row_spec(128),
            _row_spec(16),
            _full_spec(128, HID), _full_spec(128, HID),
            _full_spec(128, HID), _full_spec(128, HID),
            _full_spec(1, HID), _full_spec(HID, CLS),
        ],
        out_specs=[_row_spec(HID), _row_spec(32), _row_spec(32)],
        out_shape=[
            jax.ShapeDtypeStruct((N_PAD, HID), f32),
            jax.ShapeDtypeStruct((N_PAD, 32), jnp.bfloat16),
            jax.ShapeDtypeStruct((N_PAD, 32), jnp.bfloat16),
        ],
    )(h1a, h1b, agg1[:N_PAD], agg1[N_PAD:], degs,
      W_self1[:128], W_self1[128:], W_neigh1[:128], W_neigh1[128:],
      b1.reshape(1, HID), W_neigh2)

    table2 = jnp.concatenate([pa, pb], axis=0)
    agg2 = _seg_sum_sc(32, 16, 10, bf16)(table2, srcp, dstp, _zr(32))

    out = pl.pallas_call(
        _tc2_body,
        grid=(nb,),
        in_specs=[
            _row_spec(HID), _row_spec(32), _row_spec(32), _row_spec(16),
            _full_spec(HID, CLS), _full_spec(1, CLS),
        ],
        out_specs=_row_spec(CLS),
        out_shape=jax.ShapeDtypeStruct((N_PAD, CLS), f32),
    )(h2, agg2[:N_PAD], agg2[N_PAD:], degs, W_self2, b2.reshape(1, CLS))

    return out[:N]


# speedup vs baseline: 1.3241x; 1.3241x over previous
import functools

import jax
import jax.numpy as jnp
from jax import lax
from jax.experimental import pallas as pl
from jax.experimental.pallas import tpu as pltpu
from jax.experimental.pallas import tpu_sc as plsc

N = 10000
E = 320000
IN, HID, CLS = 128, 256, 64

NTILES = 16
N_PAD = 10240
E_PAD = 327680
ZR = 32
RPT = N_PAD // NTILES


def _seg_sum_sc(dh, sub, ch, dtype=jnp.float32):
    assert ch % 2 == 0
    mesh = plsc.VectorSubcoreMesh(
        core_axis_name="c", subcore_axis_name="s", num_cores=2,
        num_subcores=NTILES,
    )

    @functools.partial(
        pl.kernel,
        out_type=jax.ShapeDtypeStruct((2 * N_PAD, dh), dtype),
        mesh=mesh,
        compiler_params=pltpu.CompilerParams(use_tc_tiling_on_sc=False),
        scratch_types=[
            pltpu.VMEM((2, sub * 128), jnp.int32),
            pltpu.VMEM((sub * 128,), jnp.int32),
            pltpu.VMEM((sub * 128,), jnp.int32),
            pltpu.VMEM((2, sub * 128, dh), dtype),
            pltpu.VMEM((ZR, dh), dtype),
            pltpu.SemaphoreType.DMA,
            pltpu.SemaphoreType.DMA,
            pltpu.VMEM_SHARED((N_PAD, dh), dtype),
        ],
    )
    def seg_sum(table, src3, dst3, zrows, out, sidx, didx0, didx1, rows, zbuf,
                sem0, sem1, acc):
        c = lax.axis_index("c")
        s = lax.axis_index("s")
        sems = (sem0, sem1)
        didxs = (didx0, didx1)

        pltpu.sync_copy(zrows, zbuf)
        for t in range(RPT // ZR):
            pltpu.sync_copy(zbuf, acc.at[pl.ds(s * RPT + t * ZR, ZR)])
        plsc.subcore_barrier()

        off = c * N_PAD

        def stage(ci, b):
            t = s * (ch + 2) + ci
            pltpu.sync_copy(src3.at[t], sidx.at[b])
            pltpu.sync_copy(dst3.at[t], didxs[b])
            for i in range(sub * 8):
                sidx[b, pl.ds(i * 16, 16)] = sidx[b, pl.ds(i * 16, 16)] + off
            pltpu.async_copy(table.at[sidx.at[b]], rows.at[b], sems[b])

        def drain_scatter(ci, b):
            pltpu.make_async_copy(
                table.at[sidx.at[b]], rows.at[b], sems[b]
            ).wait()
            pltpu.sync_copy(rows.at[b], acc.at[didxs[b]], add=True)
            stage(ci, b)

        stage(0, 0)
        stage(1, 1)

        def body2(i2, carry):
            ci = 2 * i2
            drain_scatter(ci + 2, 0)
            drain_scatter(ci + 3, 1)
            return carry

        lax.fori_loop(0, ch // 2, body2, 0)
        for b in range(2):
            pltpu.make_async_copy(
                table.at[sidx.at[b]], rows.at[b], sems[b]
            ).wait()
        plsc.subcore_barrier()
        pltpu.sync_copy(
            acc.at[pl.ds(s * RPT, RPT)], out.at[pl.ds(off + s * RPT, RPT)]
        )

    return seg_sum


_R = 1024


def _tc0_body(x_r, a0_r, a1_r, ws_r, wa_r, wb_r, b_r, o0_r, o1_r):
    inv = 1.0 / jnp.maximum(a0_r[:, 64:65].astype(jnp.float32), 1.0)
    z = jnp.dot(x_r[...], ws_r[...], preferred_element_type=jnp.float32)
    z = z + jnp.dot(a0_r[:, :64].astype(jnp.float32) * inv, wa_r[...],
                    preferred_element_type=jnp.float32)
    z = z + jnp.dot(a1_r[:, :64].astype(jnp.float32) * inv, wb_r[...],
                    preferred_element_type=jnp.float32)
    z = jnp.maximum(z + b_r[...], 0.0)
    o0_r[...] = z[:, :128].astype(jnp.bfloat16)
    o1_r[...] = z[:, 128:].astype(jnp.bfloat16)


def _tc1_body(h0_r, h1_r, a0_r, a1_r, dg_r, wsa_r, wsb_r, wna_r, wnb_r, b_r,
              wn2_r, h2_r, p0_r, p1_r):
    inv = 1.0 / jnp.maximum(dg_r[:, :1].astype(jnp.float32), 1.0)
    z = jnp.dot(h0_r[...], wsa_r[...], preferred_element_type=jnp.float32)
    z = z + jnp.dot(h1_r[...], wsb_r[...], preferred_element_type=jnp.float32)
    z = z + jnp.dot(a0_r[...] * inv, wna_r[...], preferred_element_type=jnp.float32)
    z = z + jnp.dot(a1_r[...] * inv, wnb_r[...], preferred_element_type=jnp.float32)
    z = jnp.maximum(z + b_r[...], 0.0)
    h2_r[...] = z
    p = jnp.dot(z, wn2_r[...], preferred_element_type=jnp.float32)
    p0_r[...] = p[:, :32].astype(jnp.bfloat16)
    p1_r[...] = p[:, 32:].astype(jnp.bfloat16)


def _tc2_body(h2_r, a0_r, a1_r, dg_r, ws_r, b_r, o_r):
    inv = 1.0 / jnp.maximum(dg_r[:, :1].astype(jnp.float32), 1.0)
    m = jnp.concatenate([a0_r[...] * inv, a1_r[...] * inv], axis=1)
    z = jnp.dot(h2_r[...], ws_r[...], preferred_element_type=jnp.float32)
    o_r[...] = z + m + b_r[...]


def _row_spec(w):
    return pl.BlockSpec((_R, w), lambda i: (i, 0))


def _full_spec(h, w):
    return pl.BlockSpec((h, w), lambda i: (0, 0))


def kernel(x, edge_index, W_self0, W_neigh0, b0, W_self1, W_neigh1, b1,
           W_self2, W_neigh2, b2):
    f32 = jnp.float32
    src = edge_index[0]
    dst = edge_index[1]
    padlen = E_PAD - E
    padv = jnp.full((padlen,), N_PAD - 1, jnp.int32)
    srcf = jnp.concatenate([src, padv])
    dstf = jnp.concatenate([dst, padv])

    def _chunked(flat, sub, ch):
        arr = flat.reshape(NTILES, ch, sub * 128)
        padc = jnp.full((NTILES, 2, sub * 128), N_PAD - 1, jnp.int32)
        return jnp.concatenate([arr, padc], axis=1).reshape(-1, sub * 128)

    src0, dst0 = _chunked(srcf, 2, 80), _chunked(dstf, 2, 80)
    src1, dst1 = _chunked(srcf, 2, 80), _chunked(dstf, 2, 80)
    srcp, dstp = _chunked(srcf, 8, 20), _chunked(dstf, 8, 20)

    bf16 = jnp.bfloat16
    xp = jnp.zeros((N_PAD, IN), f32).at[:N].set(x)
    xb = xp.astype(bf16)
    onescol = jnp.zeros((N_PAD, 16), bf16).at[:N].set(1.0)
    zerocol = jnp.zeros((N_PAD, 16), bf16)
    table0 = jnp.concatenate(
        [
            jnp.concatenate([xb[:, :64], onescol], axis=1),
            jnp.concatenate([xb[:, 64:], zerocol], axis=1),
        ],
        axis=0,
    )

    def _zr(dh):
        return jnp.zeros((ZR, dh), bf16)

    agg0 = _seg_sum_sc(80, 2, 80, bf16)(table0, src0, dst0, _zr(80))
    degs = agg0[:N_PAD, 64:80]

    nb = N_PAD // _R
    h1a, h1b = pl.pallas_call(
        _tc0_body,
        grid=(nb,),
        in_specs=[
            _row_spec(IN), _row_spec(80), _row_spec(80),
            _full_spec(IN, HID), _full_spec(64, HID), _full_spec(64, HID),
            _full_spec(1, HID),
        ],
        out_specs=[_row_spec(128), _row_spec(128)],
        out_shape=[
            jax.ShapeDtypeStruct((N_PAD, 128), jnp.bfloat16),
            jax.ShapeDtypeStruct((N_PAD, 128), jnp.bfloat16),
        ],
    )(xp, agg0[:N_PAD], agg0[N_PAD:], W_self0, W_neigh0[:64], W_neigh0[64:],
      b0.reshape(1, HID))

    table1 = jnp.concatenate([h1a, h1b], axis=0)
    agg1 = _seg_sum_sc(128, 2, 80, bf16)(table1, src1, dst1, _zr(128))

    h2, pa, pb = pl.pallas_call(
        _tc1_body,
        grid=(nb,),
        in_specs=[
            _row_spec(128), _row_spec(128), _row_spec(128), _row_spec(128),
            _row_spec(16),
            _full_spec(128, HID), _full_spec(128, HID),
            _full_spec(128, HID), _full_spec(128, HID),
            _full_spec(1, HID), _full_spec(HID, CLS),
        ],
        out_specs=[_row_spec(HID), _row_spec(32), _row_spec(32)],
        out_shape=[
            jax.ShapeDtypeStruct((N_PAD, HID), f32),
            jax.ShapeDtypeStruct((N_PAD, 32), jnp.bfloat16),
            jax.ShapeDtypeStruct((N_PAD, 32), jnp.bfloat16),
        ],
    )(h1a, h1b, agg1[:N_PAD], agg1[N_PAD:], degs,
      W_self1[:128], W_self1[128:], W_neigh1[:128], W_neigh1[128:],
      b1.reshape(1, HID), W_neigh2)

    table2 = jnp.concatenate([pa, pb], axis=0)
    agg2 = _seg_sum_sc(32, 8, 20, bf16)(table2, srcp, dstp, _zr(32))

    out = pl.pallas_call(
        _tc2_body,
        grid=(nb,),
        in_specs=[
            _row_spec(HID), _row_spec(32), _row_spec(32), _row_spec(16),
            _full_spec(HID, CLS), _full_spec(1, CLS),
        ],
        out_specs=_row_spec(CLS),
        out_shape=jax.ShapeDtypeStruct((N_PAD, CLS), f32),
    )(h2, agg2[:N_PAD], agg2[N_PAD:], degs, W_self2, b2.reshape(1, CLS))

    return out[:N]


# speedup vs baseline: 1.4828x; 1.1199x over previous
import functools

import jax
import jax.numpy as jnp
from jax import lax
from jax.experimental import pallas as pl
from jax.experimental.pallas import tpu as pltpu
from jax.experimental.pallas import tpu_sc as plsc

N = 10000
E = 320000
IN, HID, CLS = 128, 256, 64

NTILES = 16
N_PAD = 10240
E_PAD = 327680
ZR = 32
RPT = N_PAD // NTILES


def _seg_sum_sc(dh, sub, ch, dtype=jnp.float32):
    assert ch % 2 == 0
    mesh = plsc.VectorSubcoreMesh(
        core_axis_name="c", subcore_axis_name="s", num_cores=2,
        num_subcores=NTILES,
    )

    @functools.partial(
        pl.kernel,
        out_type=jax.ShapeDtypeStruct((2 * N_PAD, dh), dtype),
        mesh=mesh,
        compiler_params=pltpu.CompilerParams(use_tc_tiling_on_sc=False),
        scratch_types=[
            pltpu.VMEM((2, sub * 128), jnp.int32),
            pltpu.VMEM((sub * 128,), jnp.int32),
            pltpu.VMEM((sub * 128,), jnp.int32),
            pltpu.VMEM((2, sub * 128, dh), dtype),
            pltpu.VMEM((ZR, dh), dtype),
            pltpu.SemaphoreType.DMA,
            pltpu.SemaphoreType.DMA,
            pltpu.VMEM_SHARED((N_PAD, dh), dtype),
        ],
    )
    def seg_sum(table, src3, dst3, zrows, out, sidx, didx0, didx1, rows, zbuf,
                sem0, sem1, acc):
        c = lax.axis_index("c")
        s = lax.axis_index("s")
        sems = (sem0, sem1)
        didxs = (didx0, didx1)

        pltpu.sync_copy(zrows, zbuf)
        for t in range(RPT // ZR):
            pltpu.sync_copy(zbuf, acc.at[pl.ds(s * RPT + t * ZR, ZR)])
        plsc.subcore_barrier()

        off = c * N_PAD

        def stage(ci, b):
            t = s * (ch + 2) + ci
            pltpu.sync_copy(src3.at[t], sidx.at[b])
            pltpu.sync_copy(dst3.at[t], didxs[b])
            for i in range(sub * 8):
                sidx[b, pl.ds(i * 16, 16)] = sidx[b, pl.ds(i * 16, 16)] + off
            pltpu.async_copy(table.at[sidx.at[b]], rows.at[b], sems[b])

        def drain_scatter(ci, b):
            pltpu.make_async_copy(
                table.at[sidx.at[b]], rows.at[b], sems[b]
            ).wait()
            pltpu.sync_copy(rows.at[b], acc.at[didxs[b]], add=True)
            stage(ci, b)

        stage(0, 0)
        stage(1, 1)

        def body2(i2, carry):
            ci = 2 * i2
            drain_scatter(ci + 2, 0)
            drain_scatter(ci + 3, 1)
            return carry

        lax.fori_loop(0, ch // 2, body2, 0)
        for b in range(2):
            pltpu.make_async_copy(
                table.at[sidx.at[b]], rows.at[b], sems[b]
            ).wait()
        plsc.subcore_barrier()
        pltpu.sync_copy(
            acc.at[pl.ds(s * RPT, RPT)], out.at[pl.ds(off + s * RPT, RPT)]
        )

    return seg_sum


_R = 1024


def _tc0_body(x_r, a0_r, a1_r, ws_r, wa_r, wb_r, b_r, o0_r, o1_r):
    inv = 1.0 / jnp.maximum(a0_r[:, 64:65].astype(jnp.float32), 1.0)
    z = jnp.dot(x_r[...], ws_r[...], preferred_element_type=jnp.float32)
    z = z + jnp.dot(a0_r[:, :64].astype(jnp.float32) * inv, wa_r[...],
                    preferred_element_type=jnp.float32)
    z = z + jnp.dot(a1_r[:, :64].astype(jnp.float32) * inv, wb_r[...],
                    preferred_element_type=jnp.float32)
    z = jnp.maximum(z + b_r[...], 0.0)
    o0_r[...] = z[:, :128].astype(jnp.bfloat16)
    o1_r[...] = z[:, 128:].astype(jnp.bfloat16)


def _tc1_body(h0_r, h1_r, a0_r, a1_r, dg_r, wsa_r, wsb_r, wna_r, wnb_r, b_r,
              wn2_r, h2_r, p0_r, p1_r):
    inv = 1.0 / jnp.maximum(dg_r[:, :1].astype(jnp.float32), 1.0)
    z = jnp.dot(h0_r[...], wsa_r[...], preferred_element_type=jnp.float32)
    z = z + jnp.dot(h1_r[...], wsb_r[...], preferred_element_type=jnp.float32)
    z = z + jnp.dot(a0_r[...] * inv, wna_r[...], preferred_element_type=jnp.float32)
    z = z + jnp.dot(a1_r[...] * inv, wnb_r[...], preferred_element_type=jnp.float32)
    z = jnp.maximum(z + b_r[...], 0.0)
    h2_r[...] = z
    p = jnp.dot(z, wn2_r[...], preferred_element_type=jnp.float32)
    p0_r[...] = p[:, :32].astype(jnp.bfloat16)
    p1_r[...] = p[:, 32:].astype(jnp.bfloat16)


def _tc2_body(h2_r, a0_r, a1_r, dg_r, ws_r, b_r, o_r):
    inv = 1.0 / jnp.maximum(dg_r[:, :1].astype(jnp.float32), 1.0)
    m = jnp.concatenate([a0_r[...] * inv, a1_r[...] * inv], axis=1)
    z = jnp.dot(h2_r[...], ws_r[...], preferred_element_type=jnp.float32)
    o_r[...] = z + m + b_r[...]


def _row_spec(w):
    return pl.BlockSpec((_R, w), lambda i: (i, 0))


def _full_spec(h, w):
    return pl.BlockSpec((h, w), lambda i: (0, 0))


def kernel(x, edge_index, W_self0, W_neigh0, b0, W_self1, W_neigh1, b1,
           W_self2, W_neigh2, b2):
    f32 = jnp.float32
    src = edge_index[0]
    dst = edge_index[1]
    padlen = E_PAD - E
    padv = jnp.full((padlen,), N_PAD - 1, jnp.int32)
    srcf = jnp.concatenate([src, padv])
    dstf = jnp.concatenate([dst, padv])

    def _chunked(flat, sub, ch):
        arr = flat.reshape(NTILES, ch, sub * 128)
        padc = jnp.full((NTILES, 2, sub * 128), N_PAD - 1, jnp.int32)
        return jnp.concatenate([arr, padc], axis=1).reshape(-1, sub * 128)

    src0, dst0 = _chunked(srcf, 1, 160), _chunked(dstf, 1, 160)
    src1, dst1 = _chunked(srcf, 1, 160), _chunked(dstf, 1, 160)
    srcp, dstp = _chunked(srcf, 4, 40), _chunked(dstf, 4, 40)

    bf16 = jnp.bfloat16
    xp = jnp.zeros((N_PAD, IN), f32).at[:N].set(x)
    xb = xp.astype(bf16)
    onescol = jnp.zeros((N_PAD, 16), bf16).at[:N].set(1.0)
    zerocol = jnp.zeros((N_PAD, 16), bf16)
    table0 = jnp.concatenate(
        [
            jnp.concatenate([xb[:, :64], onescol], axis=1),
            jnp.concatenate([xb[:, 64:], zerocol], axis=1),
        ],
        axis=0,
    )

    def _zr(dh):
        return jnp.zeros((ZR, dh), bf16)

    agg0 = _seg_sum_sc(80, 1, 160, bf16)(table0, src0, dst0, _zr(80))
    degs = agg0[:N_PAD, 64:80]

    nb = N_PAD // _R
    h1a, h1b = pl.pallas_call(
        _tc0_body,
        grid=(nb,),
        in_specs=[
            _row_spec(IN), _row_spec(80), _row_spec(80),
            _full_spec(IN, HID), _full_spec(64, HID), _full_spec(64, HID),
            _full_spec(1, HID),
        ],
        out_specs=[_row_spec(128), _row_spec(128)],
        out_shape=[
            jax.ShapeDtypeStruct((N_PAD, 128), jnp.bfloat16),
            jax.ShapeDtypeStruct((N_PAD, 128), jnp.bfloat16),
        ],
    )(xp, agg0[:N_PAD], agg0[N_PAD:], W_self0, W_neigh0[:64], W_neigh0[64:],
      b0.reshape(1, HID))

    table1 = jnp.concatenate([h1a, h1b], axis=0)
    agg1 = _seg_sum_sc(128, 1, 160, bf16)(table1, src1, dst1, _zr(128))

    h2, pa, pb = pl.pallas_call(
        _tc1_body,
        grid=(nb,),
        in_specs=[
            _row_spec(128), _row_spec(128), _row_spec(128), _row_spec(128),
            _row_spec(16),
            _full_spec(128, HID), _full_spec(128, HID),
            _full_spec(128, HID), _full_spec(128, HID),
            _full_spec(1, HID), _full_spec(HID, CLS),
        ],
        out_specs=[_row_spec(HID), _row_spec(32), _row_spec(32)],
        out_shape=[
            jax.ShapeDtypeStruct((N_PAD, HID), f32),
            jax.ShapeDtypeStruct((N_PAD, 32), jnp.bfloat16),
            jax.ShapeDtypeStruct((N_PAD, 32), jnp.bfloat16),
        ],
    )(h1a, h1b, agg1[:N_PAD], agg1[N_PAD:], degs,
      W_self1[:128], W_self1[128:], W_neigh1[:128], W_neigh1[128:],
      b1.reshape(1, HID), W_neigh2)

    table2 = jnp.concatenate([pa, pb], axis=0)
    agg2 = _seg_sum_sc(32, 4, 40, bf16)(table2, srcp, dstp, _zr(32))

    out = pl.pallas_call(
        _tc2_body,
        grid=(nb,),
        in_specs=[
            _row_spec(HID), _row_spec(32), _row_spec(32), _row_spec(16),
            _full_spec(HID, CLS), _full_spec(1, CLS),
        ],
        out_specs=_row_spec(CLS),
        out_shape=jax.ShapeDtypeStruct((N_PAD, CLS), f32),
    )(h2, agg2[:N_PAD], agg2[N_PAD:], degs, W_self2, b2.reshape(1, CLS))

    return out[:N]


# speedup vs baseline: 1.4929x; 1.0068x over previous
import functools

import jax
import jax.numpy as jnp
from jax import lax
from jax.experimental import pallas as pl
from jax.experimental.pallas import tpu as pltpu
from jax.experimental.pallas import tpu_sc as plsc

N = 10000
E = 320000
IN, HID, CLS = 128, 256, 64

NTILES = 16
N_PAD = 10240
E_PAD = 327680
ZR = 32
RPT = N_PAD // NTILES


def _seg_sum_sc(dh, sub, ch, dtype=jnp.float32):
    assert ch % 2 == 0
    mesh = plsc.VectorSubcoreMesh(
        core_axis_name="c", subcore_axis_name="s", num_cores=2,
        num_subcores=NTILES,
    )

    @functools.partial(
        pl.kernel,
        out_type=jax.ShapeDtypeStruct((2 * N_PAD, dh), dtype),
        mesh=mesh,
        compiler_params=pltpu.CompilerParams(use_tc_tiling_on_sc=False),
        scratch_types=[
            pltpu.VMEM((2, sub * 128), jnp.int32),
            pltpu.VMEM((sub * 128,), jnp.int32),
            pltpu.VMEM((sub * 128,), jnp.int32),
            pltpu.VMEM((2, sub * 128, dh), dtype),
            pltpu.VMEM((ZR, dh), dtype),
            pltpu.SemaphoreType.DMA,
            pltpu.SemaphoreType.DMA,
            pltpu.VMEM_SHARED((N_PAD, dh), dtype),
        ],
    )
    def seg_sum(table, src3, dst3, zrows, out, sidx, didx0, didx1, rows, zbuf,
                sem0, sem1, acc):
        c = lax.axis_index("c")
        s = lax.axis_index("s")
        sems = (sem0, sem1)
        didxs = (didx0, didx1)

        pltpu.sync_copy(zrows, zbuf)
        for t in range(RPT // ZR):
            pltpu.sync_copy(zbuf, acc.at[pl.ds(s * RPT + t * ZR, ZR)])
        plsc.subcore_barrier()

        off = c * N_PAD

        def stage(ci, b):
            t = s * (ch + 2) + ci
            pltpu.sync_copy(src3.at[t], sidx.at[b])
            pltpu.sync_copy(dst3.at[t], didxs[b])
            for i in range(sub * 8):
                sidx[b, pl.ds(i * 16, 16)] = sidx[b, pl.ds(i * 16, 16)] + off
            pltpu.async_copy(table.at[sidx.at[b]], rows.at[b], sems[b])

        def drain_scatter(ci, b):
            pltpu.make_async_copy(
                table.at[sidx.at[b]], rows.at[b], sems[b]
            ).wait()
            pltpu.sync_copy(rows.at[b], acc.at[didxs[b]], add=True)
            stage(ci, b)

        stage(0, 0)
        stage(1, 1)

        def body2(i2, carry):
            ci = 2 * i2
            drain_scatter(ci + 2, 0)
            drain_scatter(ci + 3, 1)
            return carry

        lax.fori_loop(0, ch // 2, body2, 0)
        for b in range(2):
            pltpu.make_async_copy(
                table.at[sidx.at[b]], rows.at[b], sems[b]
            ).wait()
        plsc.subcore_barrier()
        pltpu.sync_copy(
            acc.at[pl.ds(s * RPT, RPT)], out.at[pl.ds(off + s * RPT, RPT)]
        )

    return seg_sum


_R = 1024


def _tc0_body(x_r, a0_r, a1_r, ws_r, wa_r, wb_r, b_r, o0_r, o1_r):
    inv = 1.0 / jnp.maximum(a0_r[:, 64:65].astype(jnp.float32), 1.0)
    z = jnp.dot(x_r[...], ws_r[...], preferred_element_type=jnp.float32)
    z = z + jnp.dot(a0_r[:, :64].astype(jnp.float32) * inv, wa_r[...],
                    preferred_element_type=jnp.float32)
    z = z + jnp.dot(a1_r[:, :64].astype(jnp.float32) * inv, wb_r[...],
                    preferred_element_type=jnp.float32)
    z = jnp.maximum(z + b_r[...], 0.0)
    o0_r[...] = z[:, :128].astype(jnp.bfloat16)
    o1_r[...] = z[:, 128:].astype(jnp.bfloat16)


def _tc1_body(h0_r, h1_r, a0_r, a1_r, dg_r, wsa_r, wsb_r, wna_r, wnb_r, b_r,
              wn2_r, h2_r, p0_r, p1_r):
    inv = 1.0 / jnp.maximum(dg_r[:, :1].astype(jnp.float32), 1.0)
    z = jnp.dot(h0_r[...], wsa_r[...], preferred_element_type=jnp.float32)
    z = z + jnp.dot(h1_r[...], wsb_r[...], preferred_element_type=jnp.float32)
    z = z + jnp.dot(a0_r[...] * inv, wna_r[...], preferred_element_type=jnp.float32)
    z = z + jnp.dot(a1_r[...] * inv, wnb_r[...], preferred_element_type=jnp.float32)
    z = jnp.maximum(z + b_r[...], 0.0)
    h2_r[...] = z
    p = jnp.dot(z, wn2_r[...], preferred_element_type=jnp.float32)
    p0_r[...] = p[:, :32].astype(jnp.bfloat16)
    p1_r[...] = p[:, 32:].astype(jnp.bfloat16)


def _tc2_body(h2_r, a0_r, a1_r, dg_r, ws_r, b_r, o_r):
    inv = 1.0 / jnp.maximum(dg_r[:, :1].astype(jnp.float32), 1.0)
    m = jnp.concatenate([a0_r[...] * inv, a1_r[...] * inv], axis=1)
    z = jnp.dot(h2_r[...], ws_r[...], preferred_element_type=jnp.float32)
    o_r[...] = z + m + b_r[...]


def _row_spec(w):
    return pl.BlockSpec((_R, w), lambda i: (i, 0))


def _full_spec(h, w):
    return pl.BlockSpec((h, w), lambda i: (0, 0))


def kernel(x, edge_index, W_self0, W_neigh0, b0, W_self1, W_neigh1, b1,
           W_self2, W_neigh2, b2):
    f32 = jnp.float32
    src = edge_index[0]
    dst = edge_index[1]
    padlen = E_PAD - E
    padv = jnp.full((padlen,), N_PAD - 1, jnp.int32)
    srcf = jnp.concatenate([src, padv])
    dstf = jnp.concatenate([dst, padv])

    def _chunked(flat, sub, ch):
        arr = flat.reshape(NTILES, ch, sub * 128)
        padc = jnp.full((NTILES, 2, sub * 128), N_PAD - 1, jnp.int32)
        return jnp.concatenate([arr, padc], axis=1).reshape(-1, sub * 128)

    src0, dst0 = _chunked(srcf, 1, 160), _chunked(dstf, 1, 160)
    src1, dst1 = _chunked(srcf, 1, 160), _chunked(dstf, 1, 160)
    srcp, dstp = _chunked(srcf, 2, 80), _chunked(dstf, 2, 80)

    bf16 = jnp.bfloat16
    xp = jnp.zeros((N_PAD, IN), f32).at[:N].set(x)
    xb = xp.astype(bf16)
    onescol = jnp.zeros((N_PAD, 16), bf16).at[:N].set(1.0)
    zerocol = jnp.zeros((N_PAD, 16), bf16)
    table0 = jnp.concatenate(
        [
            jnp.concatenate([xb[:, :64], onescol], axis=1),
            jnp.concatenate([xb[:, 64:], zerocol], axis=1),
        ],
        axis=0,
    )

    def _zr(dh):
        return jnp.zeros((ZR, dh), bf16)

    agg0 = _seg_sum_sc(80, 1, 160, bf16)(table0, src0, dst0, _zr(80))
    degs = agg0[:N_PAD, 64:80]

    nb = N_PAD // _R
    h1a, h1b = pl.pallas_call(
        _tc0_body,
        grid=(nb,),
        in_specs=[
            _row_spec(IN), _row_spec(80), _row_spec(80),
            _full_spec(IN, HID), _full_spec(64, HID), _full_spec(64, HID),
            _full_spec(1, HID),
        ],
        out_specs=[_row_spec(128), _row_spec(128)],
        out_shape=[
            jax.ShapeDtypeStruct((N_PAD, 128), jnp.bfloat16),
            jax.ShapeDtypeStruct((N_PAD, 128), jnp.bfloat16),
        ],
    )(xp, agg0[:N_PAD], agg0[N_PAD:], W_self0, W_neigh0[:64], W_neigh0[64:],
      b0.reshape(1, HID))

    table1 = jnp.concatenate([h1a, h1b], axis=0)
    agg1 = _seg_sum_sc(128, 1, 160, bf16)(table1, src1, dst1, _zr(128))

    h2, pa, pb = pl.pallas_call(
        _tc1_body,
        grid=(nb,),
        in_specs=[
            _row_spec(128), _row_spec(128), _row_spec(128), _row_spec(128),
            _row_spec(16),
            _full_spec(128, HID), _full_spec(128, HID),
            _full_spec(128, HID), _full_spec(128, HID),
            _full_spec(1, HID), _full_spec(HID, CLS),
        ],
        out_specs=[_row_spec(HID), _row_spec(32), _row_spec(32)],
        out_shape=[
            jax.ShapeDtypeStruct((N_PAD, HID), f32),
            jax.ShapeDtypeStruct((N_PAD, 32), jnp.bfloat16),
            jax.ShapeDtypeStruct((N_PAD, 32), jnp.bfloat16),
        ],
    )(h1a, h1b, agg1[:N_PAD], agg1[N_PAD:], degs,
      W_self1[:128], W_self1[128:], W_neigh1[:128], W_neigh1[128:],
      b1.reshape(1, HID), W_neigh2)

    table2 = jnp.concatenate([pa, pb], axis=0)
    agg2 = _seg_sum_sc(32, 2, 80, bf16)(table2, srcp, dstp, _zr(32))

    out = pl.pallas_call(
        _tc2_body,
        grid=(nb,),
        in_specs=[
            _row_spec(HID), _row_spec(32), _row_spec(32), _row_spec(16),
            _full_spec(HID, CLS), _full_spec(1, CLS),
        ],
        out_specs=_row_spec(CLS),
        out_shape=jax.ShapeDtypeStruct((N_PAD, CLS), f32),
    )(h2, agg2[:N_PAD], agg2[N_PAD:], degs, W_self2, b2.reshape(1, CLS))

    return out[:N]
